# baseline (speedup 1.0000x reference)
import functools

import jax
import jax.numpy as jnp
from jax import lax
from jax.experimental import pallas as pl
from jax.experimental.pallas import tpu as pltpu
from jax.experimental.pallas import tpu_sc as plsc

_B, _C = 16384, 1000
_N = _B * _C
_CP = 1024
_PAD_COUNT = _B * (_CP - _C)
_NBINS = 1 << 16
_NC, _NS = 2, 16
_NW = _NC * _NS
_TILE_ROWS = _CP // _NW
_ROW = _B
_VPR = _ROW // 16
_COLS_BLK = 1024
_GRID = _B // _COLS_BLK


def _keys_body(x_ref, t_ref, o_ref):
    x = x_ref[...]
    t = t_ref[...]
    neg_mask = (t < 0.0).astype(jnp.float32)
    neg_loss = jnp.maximum(-x, 0.0) - x * t + jnp.log1p(jnp.exp(-jnp.abs(x)))
    unobs = neg_mask * neg_loss
    i = lax.bitcast_convert_type(unobs, jnp.int32)
    o_ref[:_C, :] = jnp.where(i >= 0, i ^ jnp.int32(-(2**31)), ~i)
    o_ref[_C:, :] = jnp.zeros((_CP - _C, _COLS_BLK), jnp.int32)


def _loss_body(x_ref, t_ref, s_ref, o_ref):
    x = x_ref[...]
    t = t_ref[...]
    thr = s_ref[0, 0]
    use = s_ref[0, 1]
    pos_mask = (t > 0.0).astype(jnp.float32)
    neg_mask = (t < 0.0).astype(jnp.float32)
    sp = jnp.log1p(jnp.exp(-jnp.abs(x)))
    xt = x * t
    pos_loss = jnp.maximum(x, 0.0) - xt + sp
    neg_loss = jnp.maximum(-x, 0.0) - xt + sp
    unobs = neg_mask * neg_loss
    keep = jnp.where(unobs < thr, 1.0, 0.0)
    keep = jnp.where(use > 0.0, keep, 1.0)
    o_ref[...] = pos_mask * pos_loss + neg_mask * keep * neg_loss


def _hist_body(low, keys_hbm, pref_hbm, out_hbm, buf0, buf1, hist, pvec,
               sem0, sem1):
    wid = lax.axis_index("s") * _NC + lax.axis_index("c")
    row0 = wid * _TILE_ROWS
    zeros = jnp.zeros((16,), jnp.int32)
    ones = jnp.ones((16,), jnp.int32)

    @plsc.parallel_loop(0, _NBINS // 16, unroll=8)
    def _(i):
        hist[pl.ds(i * 16, 16)] = zeros

    pltpu.sync_copy(pref_hbm, pvec)
    pv = pvec[...]

    bufs, sems = (buf0, buf1), (sem0, sem1)
    handles = [None, None]
    handles[0] = pltpu.async_copy(keys_hbm.at[row0, :], buf0, sem0)
    zcount = zeros
    for c in range(_TILE_ROWS):
        cur = c % 2
        handles[cur].wait()
        if c + 1 < _TILE_ROWS:
            nxt = (c + 1) % 2
            handles[nxt] = pltpu.async_copy(
                keys_hbm.at[row0 + c + 1, :], bufs[nxt], sems[nxt])
        buf = bufs[cur]

        if low:
            hot = jnp.where(pv == 0x7FFF, jnp.int32(0xFFFF), jnp.int32(0))

            @plsc.parallel_loop(0, _VPR, unroll=8, carry=zcount)
            def zc(j, acc):
                v = buf[pl.ds(j * 16, 16)]
                hi = lax.shift_right_logical(v, 16)
                lo = jnp.bitwise_and(v, jnp.int32(0xFFFF))
                match = hi == pv
                is_hot = jnp.logical_and(match, lo == hot)
                plsc.addupdate_scatter(
                    hist, [lo], ones,
                    mask=jnp.logical_and(match, lo != hot))
                return acc + plsc.all_reduce_population_count(is_hot)
            zcount = zc
        else:
            @plsc.parallel_loop(0, _VPR, unroll=8, carry=zcount)
            def zc(j, acc):
                v = buf[pl.ds(j * 16, 16)]
                hi = lax.shift_right_logical(v, 16)
                is_pz = hi == 0x8000
                is_nz = hi == 0x7FFF
                plsc.addupdate_scatter(
                    hist, [hi], ones,
                    mask=jnp.logical_not(jnp.logical_or(is_pz, is_nz)))
                return acc + plsc.all_reduce_population_count(is_pz)
            zcount = zc

    lane0 = lax.iota(jnp.int32, 16) == 0
    if low:
        hot = jnp.where(pv == 0x7FFF, jnp.int32(0xFFFF), jnp.int32(0))
        plsc.addupdate_scatter(hist, [hot], zcount, mask=lane0)
    else:
        plsc.addupdate_scatter(
            hist, [jnp.full((16,), 0x8000, jnp.int32)], zcount, mask=lane0)
    pltpu.sync_copy(hist, out_hbm.at[wid])


def _sc_hist(low):
    mesh = plsc.VectorSubcoreMesh(
        core_axis_name="c", subcore_axis_name="s",
        num_cores=_NC, num_subcores=_NS)
    return pl.kernel(
        functools.partial(_hist_body, low),
        out_type=jax.ShapeDtypeStruct((_NW, _NBINS), jnp.int32),
        mesh=mesh,
        scratch_types=[
            pltpu.VMEM((_ROW,), jnp.int32),
            pltpu.VMEM((_ROW,), jnp.int32),
            pltpu.VMEM((_NBINS,), jnp.int32),
            pltpu.VMEM((16,), jnp.int32),
            pltpu.SemaphoreType.DMA,
            pltpu.SemaphoreType.DMA,
        ],
        name="sc_hist_lo" if low else "sc_hist_hi",
        compiler_params=pltpu.CompilerParams(needs_layout_passes=False),
    )


def _scan(hist, rank):
    cum = jnp.cumsum(hist)
    b = jnp.sum((cum <= rank).astype(jnp.int32)).astype(jnp.int32)
    below = cum[b] - hist[b]
    return b, rank - below


def kernel(input, target, llr_rel):
    x = jnp.transpose(input)
    t = jnp.transpose(target)

    j = jnp.round((1.0 - llr_rel) * float(1 << 23)).astype(jnp.int32)
    a, d = 125, 64
    q = j // d
    r = j - q * d
    k = a * q + (a * r + d - 1) // d

    blk = lambda: pl.BlockSpec((_C, _COLS_BLK), lambda i: (0, i))
    keys = pl.pallas_call(
        _keys_body,
        grid=(_GRID,),
        in_specs=[blk(), blk()],
        out_specs=pl.BlockSpec((_CP, _COLS_BLK), lambda i: (0, i)),
        out_shape=jax.ShapeDtypeStruct((_CP, _B), jnp.int32),
    )(x, t)

    rank0 = _N - jnp.clip(k, 1, _N)
    zero16 = jnp.zeros((16,), jnp.int32)
    h1 = _sc_hist(False)(keys, zero16).sum(axis=0)
    h1 = h1.at[0x7FFF].add(_CP * _B - h1.sum())
    h1 = h1.at[0].add(-_PAD_COUNT)
    b_hi, rank1 = _scan(h1, rank0)
    h2 = _sc_hist(True)(keys, jnp.full((16,), b_hi)).sum(axis=0)
    h2 = h2.at[0].add(-jnp.where(b_hi == 0, _PAD_COUNT, 0))
    b_lo, _ = _scan(h2, rank1)

    key = jnp.bitwise_or(lax.shift_left(b_hi, 16), b_lo)
    bits = jnp.where(key < 0, key ^ jnp.int32(-(2**31)), ~key)
    thr = lax.bitcast_convert_type(bits, jnp.float32)
    scal = jnp.stack([thr, (k != 0).astype(jnp.float32)]).reshape(1, 2)

    loss_t = pl.pallas_call(
        _loss_body,
        grid=(_GRID,),
        in_specs=[blk(), blk(),
                  pl.BlockSpec((1, 2), lambda i: (0, 0))],
        out_specs=blk(),
        out_shape=jax.ShapeDtypeStruct((_C, _B), jnp.float32),
    )(x, t, scal)
    return jnp.transpose(loss_t)


# speedup vs baseline: 1.7997x; 1.7997x over previous
import functools

import jax
import jax.numpy as jnp
from jax import lax
from jax.experimental import pallas as pl
from jax.experimental.pallas import tpu as pltpu
from jax.experimental.pallas import tpu_sc as plsc

_B, _C = 16384, 1000
_N = _B * _C
_CP = 1024
_PAD_COUNT = _B * (_CP - _C)
_NBINS = 1 << 16
_NC, _NS = 2, 16
_NW = _NC * _NS
_TILE_ROWS = _CP // _NW
_ROW = _B
_VPR = _ROW // 16
_COLS_BLK = 1024
_GRID = _B // _COLS_BLK


def _keys_body(x_ref, t_ref, o_ref):
    x = x_ref[...]
    t = t_ref[...]
    neg_mask = (t < 0.0).astype(jnp.float32)
    neg_loss = jnp.maximum(-x, 0.0) - x * t + jnp.log1p(jnp.exp(-jnp.abs(x)))
    unobs = neg_mask * neg_loss
    i = lax.bitcast_convert_type(unobs, jnp.int32)
    o_ref[:_C, :] = jnp.where(i >= 0, i ^ jnp.int32(-(2**31)), ~i)
    o_ref[_C:, :] = jnp.full((_CP - _C, _COLS_BLK), 0x7FFF0000, jnp.int32)


def _loss_body(x_ref, t_ref, s_ref, o_ref):
    x = x_ref[...]
    t = t_ref[...]
    thr = s_ref[0, 0]
    use = s_ref[0, 1]
    sp = jnp.log1p(jnp.exp(-jnp.abs(x)))
    base = jnp.maximum(jnp.where(t > 0.0, x, -x), 0.0) - x * t + sp
    keep = jnp.logical_or(base < thr, use <= 0.0)
    w = jnp.where(t > 0.0, 1.0,
                  jnp.where(jnp.logical_and(t < 0.0, keep), 1.0, 0.0))
    o_ref[...] = base * w


def _hist_body(low, keys_hbm, pref_hbm, out_hbm, buf0, buf1, hist, pvec,
               sem0, sem1):
    wid = lax.axis_index("s") * _NC + lax.axis_index("c")
    row0 = wid * _TILE_ROWS
    zeros = jnp.zeros((16,), jnp.int32)
    ones = jnp.ones((16,), jnp.int32)

    @plsc.parallel_loop(0, _NBINS // 16, unroll=8)
    def _(i):
        hist[pl.ds(i * 16, 16)] = zeros

    pltpu.sync_copy(pref_hbm, pvec)
    pv = pvec[...]
    p0 = pv[0]
    slow = jnp.logical_or(p0 == 0x7FFF, p0 == 0x8000)

    bufs, sems = (buf0, buf1), (sem0, sem1)
    handles = [None, None]
    handles[0] = pltpu.async_copy(keys_hbm.at[row0, :], buf0, sem0)
    zcount = zeros
    for c in range(_TILE_ROWS):
        cur = c % 2
        handles[cur].wait()
        if c + 1 < _TILE_ROWS:
            nxt = (c + 1) % 2
            handles[nxt] = pltpu.async_copy(
                keys_hbm.at[row0 + c + 1, :], bufs[nxt], sems[nxt])
        buf = bufs[cur]

        if low:
            def _fast(acc):
                @plsc.parallel_loop(0, _VPR, unroll=8)
                def _(j):
                    v = buf[pl.ds(j * 16, 16)]
                    hi = lax.shift_right_logical(v, 16)
                    lo = jnp.bitwise_and(v, jnp.int32(0xFFFF))
                    plsc.addupdate_scatter(hist, [lo], ones, mask=hi == pv)
                return acc

            def _slow(acc):
                @plsc.parallel_loop(0, _VPR, unroll=8, carry=acc)
                def zc2(j, a):
                    v = buf[pl.ds(j * 16, 16)]
                    hi = lax.shift_right_logical(v, 16)
                    lo = jnp.bitwise_and(v, jnp.int32(0xFFFF))
                    match = hi == pv
                    is_z = jnp.logical_and(match, lo == 0)
                    cold = jnp.logical_and(lo != 0, lo != jnp.int32(0xFFFF))
                    plsc.addupdate_scatter(
                        hist, [lo], ones,
                        mask=jnp.logical_and(match, cold))
                    return a + plsc.all_reduce_population_count(is_z)
                return zc2

            zcount = lax.cond(slow, _slow, _fast, zcount)
        else:
            @plsc.parallel_loop(0, _VPR, unroll=8, carry=zcount)
            def zc(j, acc):
                v = buf[pl.ds(j * 16, 16)]
                hi = lax.shift_right_logical(v, 16)
                is_pz = hi == 0x8000
                is_nz = hi == 0x7FFF
                plsc.addupdate_scatter(
                    hist, [hi], ones,
                    mask=jnp.logical_not(jnp.logical_or(is_pz, is_nz)))
                return acc + plsc.all_reduce_population_count(is_pz)
            zcount = zc

    lane0 = lax.iota(jnp.int32, 16) == 0
    if low:
        plsc.addupdate_scatter(hist, [zeros], zcount, mask=lane0)
    else:
        plsc.addupdate_scatter(
            hist, [jnp.full((16,), 0x8000, jnp.int32)], zcount, mask=lane0)
    pltpu.sync_copy(hist, out_hbm.at[wid])


def _sc_hist(low):
    mesh = plsc.VectorSubcoreMesh(
        core_axis_name="c", subcore_axis_name="s",
        num_cores=_NC, num_subcores=_NS)
    return pl.kernel(
        functools.partial(_hist_body, low),
        out_type=jax.ShapeDtypeStruct((_NW, _NBINS), jnp.int32),
        mesh=mesh,
        scratch_types=[
            pltpu.VMEM((_ROW,), jnp.int32),
            pltpu.VMEM((_ROW,), jnp.int32),
            pltpu.VMEM((_NBINS,), jnp.int32),
            pltpu.VMEM((16,), jnp.int32),
            pltpu.SemaphoreType.DMA,
            pltpu.SemaphoreType.DMA,
        ],
        name="sc_hist_lo" if low else "sc_hist_hi",
        compiler_params=pltpu.CompilerParams(needs_layout_passes=False),
    )


def _scan(hist, rank):
    cum = jnp.cumsum(hist)
    b = jnp.sum((cum <= rank).astype(jnp.int32)).astype(jnp.int32)
    below = cum[b] - hist[b]
    return b, rank - below


def kernel(input, target, llr_rel):
    x = jnp.transpose(input)
    t = jnp.transpose(target)

    j = jnp.round((1.0 - llr_rel) * float(1 << 23)).astype(jnp.int32)
    a, d = 125, 64
    q = j // d
    r = j - q * d
    k = a * q + (a * r + d - 1) // d

    blk = lambda: pl.BlockSpec((_C, _COLS_BLK), lambda i: (0, i))
    keys = pl.pallas_call(
        _keys_body,
        grid=(_GRID,),
        in_specs=[blk(), blk()],
        out_specs=pl.BlockSpec((_CP, _COLS_BLK), lambda i: (0, i)),
        out_shape=jax.ShapeDtypeStruct((_CP, _B), jnp.int32),
    )(x, t)

    rank0 = _N - jnp.clip(k, 1, _N)
    zero16 = jnp.zeros((16,), jnp.int32)
    h1 = _sc_hist(False)(keys, zero16).sum(axis=0)
    h1 = h1.at[0x7FFF].add(_CP * _B - h1.sum() - _PAD_COUNT)
    b_hi, rank1 = _scan(h1, rank0)
    h2 = _sc_hist(True)(keys, jnp.full((16,), b_hi)).sum(axis=0)
    slow = jnp.logical_or(b_hi == 0x7FFF, b_hi == 0x8000)
    h2 = h2.at[0].add(-jnp.where(b_hi == 0x7FFF, _PAD_COUNT, 0))
    h2 = h2.at[0xFFFF].add(jnp.where(slow, h1[b_hi] - h2.sum(), 0))
    b_lo, _ = _scan(h2, rank1)

    key = jnp.bitwise_or(lax.shift_left(b_hi, 16), b_lo)
    bits = jnp.where(key < 0, key ^ jnp.int32(-(2**31)), ~key)
    thr = lax.bitcast_convert_type(bits, jnp.float32)
    scal = jnp.stack([thr, (k != 0).astype(jnp.float32)]).reshape(1, 2)

    loss_t = pl.pallas_call(
        _loss_body,
        grid=(_GRID,),
        in_specs=[blk(), blk(),
                  pl.BlockSpec((1, 2), lambda i: (0, 0))],
        out_specs=blk(),
        out_shape=jax.ShapeDtypeStruct((_C, _B), jnp.float32),
    )(x, t, scal)
    return jnp.transpose(loss_t)


# speedup vs baseline: 2.1336x; 1.1855x over previous
import functools

import jax
import jax.numpy as jnp
from jax import lax
from jax.experimental import pallas as pl
from jax.experimental.pallas import tpu as pltpu
from jax.experimental.pallas import tpu_sc as plsc

_B, _C = 16384, 1000
_N = _B * _C
_CP = 1024
_PAD_COUNT = _B * (_CP - _C)
_NBINS = 1 << 16
_NC, _NS = 2, 16
_NW = _NC * _NS
_TILE_ROWS = _CP // _NW
_ROW = _B
_VPR = _ROW // 16
_COLS_BLK = 1024
_GRID = _B // _COLS_BLK


def _keys_body(x_ref, t_ref, o_ref):
    x = x_ref[...]
    t = t_ref[...]
    neg_mask = (t < 0.0).astype(jnp.float32)
    neg_loss = jnp.maximum(-x, 0.0) - x * t + jnp.log1p(jnp.exp(-jnp.abs(x)))
    unobs = neg_mask * neg_loss
    i = lax.bitcast_convert_type(unobs, jnp.int32)
    o_ref[:_C, :] = jnp.where(i >= 0, i ^ jnp.int32(-(2**31)), ~i)
    o_ref[_C:, :] = jnp.full((_CP - _C, _COLS_BLK), 0x7FFF0000, jnp.int32)


def _loss_body(x_ref, t_ref, s_ref, o_ref):
    x = x_ref[...]
    t = t_ref[...]
    thr = s_ref[0, 0]
    use = s_ref[0, 1]
    sp = jnp.log1p(jnp.exp(-jnp.abs(x)))
    base = jnp.maximum(jnp.where(t > 0.0, x, -x), 0.0) - x * t + sp
    keep = jnp.logical_or(base < thr, use <= 0.0)
    on = jnp.logical_or(t > 0.0, jnp.logical_and(t < 0.0, keep))
    o_ref[...] = jnp.where(on, base, 0.0)


def _hist_body(low, keys_hbm, pref_hbm, out_hbm, buf0, buf1, hist, pvec,
               sem0, sem1):
    wid = lax.axis_index("s") * _NC + lax.axis_index("c")
    row0 = wid * _TILE_ROWS
    zeros = jnp.zeros((16,), jnp.int32)
    ones = jnp.ones((16,), jnp.int32)

    @plsc.parallel_loop(0, _NBINS // 16, unroll=8)
    def _(i):
        hist[pl.ds(i * 16, 16)] = zeros

    pltpu.sync_copy(pref_hbm, pvec)
    pv = pvec[...]
    p0 = pv[0]
    slow = jnp.logical_or(p0 == 0x7FFF, p0 == 0x8000)

    bufs, sems = (buf0, buf1), (sem0, sem1)
    handles = [None, None]
    handles[0] = pltpu.async_copy(keys_hbm.at[row0, :], buf0, sem0)
    zcount = zeros
    for c in range(_TILE_ROWS):
        cur = c % 2
        handles[cur].wait()
        if c + 1 < _TILE_ROWS:
            nxt = (c + 1) % 2
            handles[nxt] = pltpu.async_copy(
                keys_hbm.at[row0 + c + 1, :], bufs[nxt], sems[nxt])
        buf = bufs[cur]

        if low:
            def _fast(acc):
                @plsc.parallel_loop(0, _VPR, unroll=8)
                def _(j):
                    v = buf[pl.ds(j * 16, 16)]
                    hi = lax.shift_right_logical(v, 16)
                    lo = jnp.bitwise_and(v, jnp.int32(0xFFFF))
                    plsc.addupdate_scatter(hist, [lo], ones, mask=hi == pv)
                return acc

            def _slow(acc):
                @plsc.parallel_loop(0, _VPR, unroll=8, carry=acc)
                def zc2(j, a):
                    v = buf[pl.ds(j * 16, 16)]
                    hi = lax.shift_right_logical(v, 16)
                    lo = jnp.bitwise_and(v, jnp.int32(0xFFFF))
                    match = hi == pv
                    is_z = jnp.logical_and(match, lo == 0)
                    cold = jnp.logical_and(lo != 0, lo != jnp.int32(0xFFFF))
                    plsc.addupdate_scatter(
                        hist, [lo], ones,
                        mask=jnp.logical_and(match, cold))
                    return a + plsc.all_reduce_population_count(is_z)
                return zc2

            zcount = lax.cond(slow, _slow, _fast, zcount)
        else:
            @plsc.parallel_loop(0, _VPR, unroll=8, carry=zcount)
            def zc(j, acc):
                v = buf[pl.ds(j * 16, 16)]
                hi = lax.shift_right_logical(v, 16)
                is_pz = hi == 0x8000
                is_nz = hi == 0x7FFF
                plsc.addupdate_scatter(
                    hist, [hi], ones,
                    mask=jnp.logical_not(jnp.logical_or(is_pz, is_nz)))
                return acc + plsc.all_reduce_population_count(is_pz)
            zcount = zc

    lane0 = lax.iota(jnp.int32, 16) == 0
    if low:
        plsc.addupdate_scatter(hist, [zeros], zcount, mask=lane0)
    else:
        plsc.addupdate_scatter(
            hist, [jnp.full((16,), 0x8000, jnp.int32)], zcount, mask=lane0)
    pltpu.sync_copy(hist, out_hbm.at[wid])


def _sc_hist(low):
    mesh = plsc.VectorSubcoreMesh(
        core_axis_name="c", subcore_axis_name="s",
        num_cores=_NC, num_subcores=_NS)
    return pl.kernel(
        functools.partial(_hist_body, low),
        out_type=jax.ShapeDtypeStruct((_NW, _NBINS), jnp.int32),
        mesh=mesh,
        scratch_types=[
            pltpu.VMEM((_ROW,), jnp.int32),
            pltpu.VMEM((_ROW,), jnp.int32),
            pltpu.VMEM((_NBINS,), jnp.int32),
            pltpu.VMEM((16,), jnp.int32),
            pltpu.SemaphoreType.DMA,
            pltpu.SemaphoreType.DMA,
        ],
        name="sc_hist_lo" if low else "sc_hist_hi",
        compiler_params=pltpu.CompilerParams(needs_layout_passes=False),
    )


def _scan(hist, rank):
    cum = jnp.cumsum(hist)
    b = jnp.sum((cum <= rank).astype(jnp.int32)).astype(jnp.int32)
    below = cum[b] - hist[b]
    return b, rank - below


def kernel(input, target, llr_rel):
    x = jnp.transpose(input)
    t = jnp.transpose(target)

    j = jnp.round((1.0 - llr_rel) * float(1 << 23)).astype(jnp.int32)
    a, d = 125, 64
    q = j // d
    r = j - q * d
    k = a * q + (a * r + d - 1) // d

    blk = lambda: pl.BlockSpec((_C, _COLS_BLK), lambda i: (0, i))
    keys = pl.pallas_call(
        _keys_body,
        grid=(_GRID,),
        in_specs=[blk(), blk()],
        out_specs=pl.BlockSpec((_CP, _COLS_BLK), lambda i: (0, i)),
        out_shape=jax.ShapeDtypeStruct((_CP, _B), jnp.int32),
    )(x, t)

    rank0 = _N - jnp.clip(k, 1, _N)
    zero16 = jnp.zeros((16,), jnp.int32)
    h1 = _sc_hist(False)(keys, zero16).sum(axis=0)
    h1 = h1.at[0x7FFF].add(_CP * _B - h1.sum() - _PAD_COUNT)
    b_hi, rank1 = _scan(h1, rank0)
    h2 = _sc_hist(True)(keys, jnp.full((16,), b_hi)).sum(axis=0)
    slow = jnp.logical_or(b_hi == 0x7FFF, b_hi == 0x8000)
    h2 = h2.at[0].add(-jnp.where(b_hi == 0x7FFF, _PAD_COUNT, 0))
    h2 = h2.at[0xFFFF].add(jnp.where(slow, h1[b_hi] - h2.sum(), 0))
    b_lo, _ = _scan(h2, rank1)

    key = jnp.bitwise_or(lax.shift_left(b_hi, 16), b_lo)
    bits = jnp.where(key < 0, key ^ jnp.int32(-(2**31)), ~key)
    thr = lax.bitcast_convert_type(bits, jnp.float32)
    scal = jnp.stack([thr, (k != 0).astype(jnp.float32)]).reshape(1, 2)

    loss_t = pl.pallas_call(
        _loss_body,
        grid=(_GRID,),
        in_specs=[blk(), blk(),
                  pl.BlockSpec(memory_space=pltpu.SMEM)],
        out_specs=blk(),
        out_shape=jax.ShapeDtypeStruct((_C, _B), jnp.float32),
    )(x, t, scal)
    return jnp.transpose(loss_t)
